# initial kernel scaffold (unmeasured)
import jax
import jax.numpy as jnp
from jax import lax
from jax.experimental import pallas as pl
from jax.experimental.pallas import tpu as pltpu


def kernel(
    t,
):
    def body(*refs):
        pass

    out_shape = jax.ShapeDtypeStruct(..., jnp.float32)
    return pl.pallas_call(body, out_shape=out_shape)(...)


# baseline (device time: 300992 ns/iter reference)
import jax
import jax.numpy as jnp
from jax import lax
from jax.experimental import pallas as pl
from jax.experimental.pallas import tpu as pltpu

N_DEV = 32
M = 2048
N = 1024
CHUNK = M // N_DEV


def _f(s):
    r = jnp.maximum(s, 0.0)
    return jnp.tanh(s) * s * s + r * r * r


def kernel(t):
    def body(x_ref, out_ref, rs_buf, rs_recv_sems, ag_recv_sems, send_sem):
        my_pos = lax.axis_index("i")
        left = lax.rem(my_pos - 1 + N_DEV, N_DEV)
        right = lax.rem(my_pos + 1, N_DEV)

        barrier_sem = pltpu.get_barrier_semaphore()
        for nbr in (left, right):
            pl.semaphore_signal(
                barrier_sem, inc=1,
                device_id=(nbr,), device_id_type=pl.DeviceIdType.MESH,
            )
        pl.semaphore_wait(barrier_sem, 2)

        out_ref[:, :] = x_ref[:, :]

        def chunk_slice(c):
            return pl.ds(c * CHUNK, CHUNK)

        for h in range(N_DEV - 1):
            send_c = lax.rem(my_pos - h + N_DEV, N_DEV)
            recv_c = lax.rem(my_pos - 1 - h + N_DEV, N_DEV)
            rdma = pltpu.make_async_remote_copy(
                src_ref=out_ref.at[chunk_slice(send_c), :],
                dst_ref=rs_buf.at[h],
                send_sem=send_sem,
                recv_sem=rs_recv_sems.at[h],
                device_id=(right,),
                device_id_type=pl.DeviceIdType.MESH,
            )
            rdma.start()
            rdma.wait()
            out_ref[chunk_slice(recv_c), :] = (
                out_ref[chunk_slice(recv_c), :] + rs_buf[h]
            )

        red_c = right
        out_ref[chunk_slice(red_c), :] = _f(out_ref[chunk_slice(red_c), :])

        for h in range(N_DEV - 1):
            send_c = lax.rem(my_pos + 1 - h + N_DEV, N_DEV)
            rdma = pltpu.make_async_remote_copy(
                src_ref=out_ref.at[chunk_slice(send_c), :],
                dst_ref=out_ref.at[chunk_slice(send_c), :],
                send_sem=send_sem,
                recv_sem=ag_recv_sems.at[h],
                device_id=(right,),
                device_id_type=pl.DeviceIdType.MESH,
            )
            rdma.start()
            rdma.wait()

    return pl.pallas_call(
        body,
        out_shape=jax.ShapeDtypeStruct((M, N), jnp.float32),
        in_specs=[pl.BlockSpec(memory_space=pltpu.VMEM)],
        out_specs=pl.BlockSpec(memory_space=pltpu.VMEM),
        scratch_shapes=[
            pltpu.VMEM((N_DEV - 1, CHUNK, N), jnp.float32),
            pltpu.SemaphoreType.DMA((N_DEV - 1,)),
            pltpu.SemaphoreType.DMA((N_DEV - 1,)),
            pltpu.SemaphoreType.DMA,
        ],
        compiler_params=pltpu.CompilerParams(collective_id=0),
    )(t)


# device time: 225022 ns/iter; 1.3376x vs baseline; 1.3376x over previous
import jax
import jax.numpy as jnp
import numpy as np
from jax import lax
from jax.experimental import pallas as pl
from jax.experimental.pallas import tpu as pltpu

N_DEV = 32
M = 2048
N = 1024
NCHUNK = 32
CHUNK = M // NCHUNK
MAX_CH = 5
SEND_RING = 8

_PLANE = [(0, 0), (1, 0), (1, 1), (0, 1), (0, 2), (1, 2), (1, 3), (0, 3)]
_POS2COORD = {}
_COORD2POS = {}
for _z in range(4):
    for _i, (_x, _y) in enumerate(_PLANE):
        _p = 8 * _z + _i
        _POS2COORD[_p] = (_x, _y, _z)
        _COORD2POS[(_x, _y, _z)] = _p

_ROOT = (0, 1, 1)


def _parent_coord(c):
    x, y, z = c
    if c == _ROOT:
        return None
    if z < _ROOT[2]:
        return (x, y, z + 1)
    if z > _ROOT[2]:
        return (x, y, z - 1)
    if y < _ROOT[1]:
        return (x, y + 1, z)
    if y > _ROOT[1]:
        return (x, y - 1, z)
    return _ROOT


_PARENT = np.full((N_DEV,), -1, np.int32)
_CHILDREN = np.full((N_DEV, MAX_CH), -1, np.int32)
_NC = np.zeros((N_DEV,), np.int32)
_SLOT = np.zeros((N_DEV,), np.int32)
for _p in range(N_DEV):
    _pc = _parent_coord(_POS2COORD[_p])
    if _pc is None:
        continue
    _pp = _COORD2POS[_pc]
    _PARENT[_p] = _pp
    _SLOT[_p] = _NC[_pp]
    _CHILDREN[_pp, _NC[_pp]] = _p
    _NC[_pp] += 1

_META = np.concatenate(
    [_PARENT[:, None], _NC[:, None], _SLOT[:, None], _CHILDREN], axis=1
).astype(np.int32)


def _f(s):
    r = jnp.maximum(s, 0.0)
    return jnp.tanh(s) * s * s + r * r * r


def kernel(t):
    my_pos = lax.axis_index("i")
    meta = jnp.asarray(_META)[my_pos]

    def body(meta_ref, x_ref, out_ref, up_buf,
             up_recv_sems, down_recv_sems, up_send_sems, down_send_sems):
        parent = meta_ref[0]
        nc = meta_ref[1]
        my_slot = meta_ref[2]
        has_parent = parent >= 0

        def rows(c):
            return pl.ds(c * CHUNK, CHUNK)

        def up_send_desc(c):
            return pltpu.make_async_remote_copy(
                src_ref=out_ref.at[rows(c), :],
                dst_ref=up_buf.at[c, my_slot],
                send_sem=up_send_sems.at[c % SEND_RING],
                recv_sem=up_recv_sems.at[c, my_slot],
                device_id=(parent,),
                device_id_type=pl.DeviceIdType.MESH,
            )

        def up_recv_desc(c, j):
            return pltpu.make_async_remote_copy(
                src_ref=up_buf.at[c, j],
                dst_ref=up_buf.at[c, j],
                send_sem=up_send_sems.at[0],
                recv_sem=up_recv_sems.at[c, j],
                device_id=(parent,),
                device_id_type=pl.DeviceIdType.MESH,
            )

        def down_send_desc(c, j):
            return pltpu.make_async_remote_copy(
                src_ref=out_ref.at[rows(c), :],
                dst_ref=out_ref.at[rows(c), :],
                send_sem=down_send_sems.at[c % SEND_RING, j],
                recv_sem=down_recv_sems.at[c],
                device_id=(meta_ref[3 + j],),
                device_id_type=pl.DeviceIdType.MESH,
            )

        def down_recv_desc(c):
            return pltpu.make_async_remote_copy(
                src_ref=out_ref.at[rows(c), :],
                dst_ref=out_ref.at[rows(c), :],
                send_sem=up_send_sems.at[0],
                recv_sem=down_recv_sems.at[c],
                device_id=(parent,),
                device_id_type=pl.DeviceIdType.MESH,
            )

        def down_send(c):
            for j in range(MAX_CH):
                @pl.when(j < nc)
                def _():
                    if c >= SEND_RING:
                        down_send_desc(c - SEND_RING, j).wait_send()
                    down_send_desc(c, j).start()

        barrier_sem = pltpu.get_barrier_semaphore()

        @pl.when(has_parent)
        def _():
            pl.semaphore_signal(
                barrier_sem, inc=1,
                device_id=(parent,), device_id_type=pl.DeviceIdType.MESH,
            )

        for j in range(MAX_CH):
            @pl.when(j < nc)
            def _():
                pl.semaphore_signal(
                    barrier_sem, inc=1,
                    device_id=(meta_ref[3 + j],),
                    device_id_type=pl.DeviceIdType.MESH,
                )
        pl.semaphore_wait(barrier_sem, nc + jnp.where(has_parent, 1, 0))

        for c in range(NCHUNK):
            out_ref[rows(c), :] = x_ref[rows(c), :]
            for j in range(MAX_CH):
                @pl.when(j < nc)
                def _():
                    up_recv_desc(c, j).wait_recv()
                    out_ref[rows(c), :] = out_ref[rows(c), :] + up_buf[c, j]

            @pl.when(has_parent)
            def _():
                if c >= SEND_RING:
                    up_send_desc(c - SEND_RING).wait_send()
                up_send_desc(c).start()

            @pl.when(jnp.logical_not(has_parent))
            def _():
                out_ref[rows(c), :] = _f(out_ref[rows(c), :])
                down_send(c)

        for c in range(NCHUNK - SEND_RING, NCHUNK):
            @pl.when(has_parent)
            def _():
                up_send_desc(c).wait_send()

        @pl.when(has_parent)
        def _():
            for c in range(NCHUNK):
                down_recv_desc(c).wait_recv()
                down_send(c)

        for c in range(NCHUNK - SEND_RING, NCHUNK):
            for j in range(MAX_CH):
                @pl.when(j < nc)
                def _():
                    down_send_desc(c, j).wait_send()

    return pl.pallas_call(
        body,
        out_shape=jax.ShapeDtypeStruct((M, N), jnp.float32),
        in_specs=[
            pl.BlockSpec(memory_space=pltpu.SMEM),
            pl.BlockSpec(memory_space=pltpu.VMEM),
        ],
        out_specs=pl.BlockSpec(memory_space=pltpu.VMEM),
        scratch_shapes=[
            pltpu.VMEM((NCHUNK, MAX_CH, CHUNK, N), jnp.float32),
            pltpu.SemaphoreType.DMA((NCHUNK, MAX_CH)),
            pltpu.SemaphoreType.DMA((NCHUNK,)),
            pltpu.SemaphoreType.DMA((SEND_RING,)),
            pltpu.SemaphoreType.DMA((SEND_RING, MAX_CH)),
        ],
        compiler_params=pltpu.CompilerParams(
            collective_id=0,
            vmem_limit_bytes=100 * 1024 * 1024,
        ),
    )(meta, t)
